# baseline (speedup 1.0000x reference)
import functools

import jax
import jax.numpy as jnp
from jax import lax
from jax.experimental import pallas as pl
from jax.experimental.pallas import tpu as pltpu
from jax.experimental.pallas import tpu_sc as plsc

N = 50000
E = 800000
NUM_GNBS = 19
H = 64
IN_F = 6

NP = 50176
EP = 860160
NC = 2
NS = 16
NW = NC * NS
EPW = EP // NW
EPT = EP // NS
CH = 128
RPT = NP // NS

_mesh = plsc.VectorSubcoreMesh(core_axis_name="c", subcore_axis_name="s")


_NSUPW = 15


@functools.partial(
    pl.kernel, mesh=_mesh,
    compiler_params=pltpu.CompilerParams(needs_layout_passes=False, use_tc_tiling_on_sc=False),
    out_type=jax.ShapeDtypeStruct((NC * NP,), jnp.float32),
    scratch_types=[
        pltpu.VMEM_SHARED((NP,), jnp.float32),
        pltpu.VMEM((14, 128), jnp.int32),
        pltpu.VMEM((14, 128), jnp.float32),
        pltpu.VMEM((14, 128), jnp.int32),
        pltpu.VMEM((14, 128), jnp.float32),
        pltpu.VMEM((RPT,), jnp.float32),
        pltpu.SemaphoreType.DMA,
        pltpu.SemaphoreType.DMA,
        pltpu.SemaphoreType.DMA,
        pltpu.SemaphoreType.DMA,
    ],
)
def _deg_kernel(dst_h, ew_h, deg_h, acc, didxA, ewA, didxB, ewB, zb,
                sIA, sIB, sS0, sS1):
    c = lax.axis_index("c")
    s = lax.axis_index("s")
    wrk = s * NC + c
    base_rows = wrk * _NSUPW * SUBS
    def _z(i, _):
        zb[pl.ds(pl.multiple_of(i * 16, 16), 16)] = jnp.zeros((16,), jnp.float32)
        return _
    lax.fori_loop(0, RPT // 16, _z, None)
    pltpu.sync_copy(zb, acc.at[pl.ds(s * RPT, RPT)])
    plsc.subcore_barrier()

    def _issue_inputs(sup, didx, ewb, sem):
        r0 = base_rows + sup * SUBS
        pltpu.async_copy(dst_h.at[pl.ds(r0, SUBS)], didx, sem)
        pltpu.async_copy(ew_h.at[pl.ds(r0, SUBS)], ewb, sem)

    def _wait_inputs(didx, ewb, sem):
        pltpu.make_async_copy(dst_h.at[pl.ds(0, SUBS)], didx, sem).wait()
        pltpu.make_async_copy(ew_h.at[pl.ds(0, SUBS)], ewb, sem).wait()

    def _s_wait(ewb, sem):
        pltpu.make_async_copy(ewb.at[0], acc.at[pl.ds(0, 128)], sem).wait()

    def _super(sup, didxP, ewP, didxO, ewO, semO):
        @pl.when(sup < _NSUPW - 1)
        def _():
            _issue_inputs(sup + 1, didxO, ewO, semO)
        def _sub(r, _):
            @pl.when(r % 2 == 0)
            def _():
                @pl.when(r >= 2)
                def _():
                    _s_wait(ewP, sS0)
                pltpu.async_copy(ewP.at[r], acc.at[didxP.at[r]], sS0, add=True)
            @pl.when(r % 2 == 1)
            def _():
                @pl.when(r >= 2)
                def _():
                    _s_wait(ewP, sS1)
                pltpu.async_copy(ewP.at[r], acc.at[didxP.at[r]], sS1, add=True)
            return _
        lax.fori_loop(0, SUBS, _sub, None)
        _s_wait(ewP, sS0)
        _s_wait(ewP, sS1)
        @pl.when(sup < _NSUPW - 1)
        def _():
            _wait_inputs(didxO, ewO, semO)

    _issue_inputs(0, didxA, ewA, sIA)
    _wait_inputs(didxA, ewA, sIA)
    def _sloop(sup, _):
        @pl.when(sup % 2 == 0)
        def _():
            _super(sup, didxA, ewA, didxB, ewB, sIB)
        @pl.when(sup % 2 == 1)
        def _():
            _super(sup, didxB, ewB, didxA, ewA, sIA)
        return _
    lax.fori_loop(0, _NSUPW, _sloop, None)
    plsc.subcore_barrier()
    pltpu.sync_copy(acc.at[pl.ds(s * RPT, RPT)], zb)
    pltpu.sync_copy(zb, deg_h.at[pl.ds(c * NP + s * RPT, RPT)])


def _dinv_body(deg_ref, o_ref):
    o_ref[...] = lax.rsqrt(deg_ref[0] + deg_ref[1])


def _dinv(degp):
    return pl.pallas_call(
        _dinv_body,
        out_shape=jax.ShapeDtypeStruct((392, 128), jnp.float32),
    )(degp.reshape(NC, 392, 128)).reshape(NP)


@functools.partial(
    pl.kernel, mesh=_mesh,
    compiler_params=pltpu.CompilerParams(needs_layout_passes=False, use_tc_tiling_on_sc=False),
    out_type=jax.ShapeDtypeStruct((EP // 128, 128), jnp.float32),
    scratch_types=[
        pltpu.VMEM((N,), jnp.float32),
        pltpu.VMEM((14, 128), jnp.int32),
        pltpu.VMEM((14, 128), jnp.int32),
        pltpu.VMEM((14, 128), jnp.float32),
        pltpu.VMEM((14, 128), jnp.int32),
        pltpu.VMEM((14, 128), jnp.int32),
        pltpu.VMEM((14, 128), jnp.float32),
        pltpu.VMEM((14, 128), jnp.float32),
        pltpu.VMEM((14, 128), jnp.float32),
        pltpu.SemaphoreType.DMA,
        pltpu.SemaphoreType.DMA,
        pltpu.SemaphoreType.DMA,
        pltpu.SemaphoreType.DMA,
    ],
)
def _norm_kernel(src_h, dst_h, ew_h, dinv_h, norm_h, dinvb,
                 sbA, dbA, ebA, sbB, dbB, ebB, nbA, nbB,
                 sIA, sIB, sOA, sOB):
    c = lax.axis_index("c")
    s = lax.axis_index("s")
    wrk = s * NC + c
    base_rows = wrk * _NSUPW * SUBS
    pltpu.sync_copy(dinv_h.at[pl.ds(0, N)], dinvb)

    def _issue_inputs(sup, sb, db, eb, sem):
        r0 = base_rows + sup * SUBS
        pltpu.async_copy(src_h.at[pl.ds(r0, SUBS)], sb, sem)
        pltpu.async_copy(dst_h.at[pl.ds(r0, SUBS)], db, sem)
        pltpu.async_copy(ew_h.at[pl.ds(r0, SUBS)], eb, sem)

    def _wait_inputs(sb, db, eb, sem):
        pltpu.make_async_copy(src_h.at[pl.ds(0, SUBS)], sb, sem).wait()
        pltpu.make_async_copy(dst_h.at[pl.ds(0, SUBS)], db, sem).wait()
        pltpu.make_async_copy(ew_h.at[pl.ds(0, SUBS)], eb, sem).wait()

    def _o_wait(nb, sem):
        pltpu.make_async_copy(norm_h.at[pl.ds(0, SUBS)], nb, sem).wait()

    def _super(sup, P, O, semO, nbP, semOutP):
        sbP, dbP, ebP = P
        sbO, dbO, ebO = O
        @pl.when(sup < _NSUPW - 1)
        def _():
            _issue_inputs(sup + 1, sbO, dbO, ebO, semO)
        @pl.when(sup >= 2)
        def _():
            _o_wait(nbP, semOutP)
        def _sub(r, _):
            for g in range(8):
                o = g * 16
                s16 = sbP[r, pl.ds(o, 16)]
                d16 = dbP[r, pl.ds(o, 16)]
                e16 = ebP[r, pl.ds(o, 16)]
                dv_s = plsc.load_gather(dinvb, [s16])
                dv_d = plsc.load_gather(dinvb, [d16])
                nbP[r, pl.ds(o, 16)] = dv_s * e16 * dv_d
            return _
        lax.fori_loop(0, SUBS, _sub, None)
        pltpu.async_copy(nbP, norm_h.at[pl.ds(base_rows + sup * SUBS, SUBS)],
                         semOutP)
        @pl.when(sup < _NSUPW - 1)
        def _():
            _wait_inputs(sbO, dbO, ebO, semO)

    _issue_inputs(0, sbA, dbA, ebA, sIA)
    _wait_inputs(sbA, dbA, ebA, sIA)
    def _sloop(sup, _):
        @pl.when(sup % 2 == 0)
        def _():
            _super(sup, (sbA, dbA, ebA), (sbB, dbB, ebB), sIB, nbA, sOA)
        @pl.when(sup % 2 == 1)
        def _():
            _super(sup, (sbB, dbB, ebB), (sbA, dbA, ebA), sIA, nbB, sOB)
        return _
    lax.fori_loop(0, _NSUPW, _sloop, None)
    _o_wait(nbB, sOB)
    _o_wait(nbA, sOA)


SUBS = 14
PAIRS = SUBS // 2


def _make_agg(W, nsup, edge_split):
    @functools.partial(
        pl.kernel, mesh=_mesh,
        compiler_params=pltpu.CompilerParams(
            needs_layout_passes=False, use_tc_tiling_on_sc=False),
        out_type=jax.ShapeDtypeStruct((NC, NP, W), jnp.float32),
        scratch_types=[
            pltpu.VMEM_SHARED((NP, W), jnp.float32),
            pltpu.VMEM((SUBS, 128), jnp.int32),
            pltpu.VMEM((SUBS, 128), jnp.int32),
            pltpu.VMEM((SUBS, 128), jnp.float32),
            pltpu.VMEM((SUBS, 128), jnp.int32),
            pltpu.VMEM((SUBS, 128), jnp.int32),
            pltpu.VMEM((SUBS, 128), jnp.float32),
            pltpu.VMEM((128, W), jnp.float32),
            pltpu.VMEM((128, W), jnp.float32),
            pltpu.VMEM((64, W), jnp.float32),
            pltpu.SemaphoreType.DMA,
            pltpu.SemaphoreType.DMA,
            pltpu.SemaphoreType.DMA,
            pltpu.SemaphoreType.DMA,
            pltpu.SemaphoreType.DMA,
            pltpu.SemaphoreType.DMA,
        ],
    )
    def _k(src_h, dst_h, norm_h, tab_h, agg_h,
           acc, sidxA, didxA, nbA, sidxB, didxB, nbB, g0, g1, zb,
           sIA, sIB, sG0, sG1, sS0, sS1):
        c = lax.axis_index("c")
        s = lax.axis_index("s")
        if edge_split:
            wrk = s * NC + c
        else:
            wrk = s
        base_rows = wrk * nsup * SUBS
        roffv = jnp.zeros((16,), jnp.int32) + c * NP

        for i in range(64):
            for q in range(W // 16):
                zb[i, pl.ds(q * 16, 16)] = jnp.zeros((16,), jnp.float32)
        def _zl(k2, _):
            pltpu.sync_copy(zb, acc.at[pl.ds(s * RPT + k2 * 64, 64)])
            return _
        lax.fori_loop(0, RPT // 64, _zl, None)
        plsc.subcore_barrier()

        def _issue_inputs(sup, sidx, didx, nb, sem):
            r0 = base_rows + sup * SUBS
            pltpu.async_copy(src_h.at[pl.ds(r0, SUBS)], sidx, sem)
            pltpu.async_copy(dst_h.at[pl.ds(r0, SUBS)], didx, sem)
            pltpu.async_copy(norm_h.at[pl.ds(r0, SUBS)], nb, sem)

        def _wait_inputs(sidx, didx, nb, sem):
            pltpu.make_async_copy(src_h.at[pl.ds(0, SUBS)], sidx, sem).wait()
            pltpu.make_async_copy(dst_h.at[pl.ds(0, SUBS)], didx, sem).wait()
            pltpu.make_async_copy(norm_h.at[pl.ds(0, SUBS)], nb, sem).wait()

        def _offsets(sidx):
            if edge_split:
                return
            def _orow(r, _):
                def _og(g, _2):
                    o = pl.multiple_of(g * 16, 16)
                    sidx[r, pl.ds(o, 16)] = sidx[r, pl.ds(o, 16)] + roffv
                    return _2
                lax.fori_loop(0, 8, _og, None)
                return _
            lax.fori_loop(0, SUBS, _orow, None)

        def _g_issue(sidx, row, gb, sem):
            pltpu.async_copy(tab_h.at[sidx.at[row]], gb, sem)

        def _g_wait(gb, sem):
            pltpu.make_async_copy(tab_h.at[pl.ds(0, 128)], gb, sem).wait()

        def _s_issue(didx, row, gb, sem):
            pltpu.async_copy(gb, acc.at[didx.at[row]], sem, add=True)

        def _s_wait(gb, sem):
            pltpu.make_async_copy(gb, acc.at[pl.ds(0, 128)], sem).wait()

        def _scale(gb, nb, rowv):
            def _se(e, _):
                colv = jnp.zeros((16,), jnp.int32) + e
                spl = plsc.load_gather(nb, [rowv, colv])
                for q in range(W // 16):
                    o = q * 16
                    gb[e, pl.ds(o, 16)] = gb[e, pl.ds(o, 16)] * spl
                return _
            lax.fori_loop(0, 128, _se, None, unroll=16)

        def _super(sup, P, O, semO):
            sidxP, didxP, nbP = P
            sidxO, didxO, nbO = O
            @pl.when(sup > 0)
            def _():
                _s_wait(g1, sS1)
            @pl.when(sup < nsup - 1)
            def _():
                _issue_inputs(sup + 1, sidxO, didxO, nbO, semO)

            def _pair(t, _):
                a2 = 2 * t
                rowa = jnp.zeros((16,), jnp.int32) + a2
                _g_wait(g0, sG0)
                @pl.when(t > 0)
                def _():
                    _s_wait(g1, sS1)
                _g_issue(sidxP, a2 + 1, g1, sG1)
                _scale(g0, nbP, rowa)
                _s_issue(didxP, a2, g0, sS0)
                _g_wait(g1, sG1)
                _scale(g1, nbP, rowa + 1)
                _s_wait(g0, sS0)
                @pl.when(t < PAIRS - 1)
                def _():
                    _g_issue(sidxP, a2 + 2, g0, sG0)
                _s_issue(didxP, a2 + 1, g1, sS1)
                return _
            lax.fori_loop(0, PAIRS, _pair, None)

            @pl.when(sup < nsup - 1)
            def _():
                _wait_inputs(sidxO, didxO, nbO, semO)
                _offsets(sidxO)
                _g_issue(sidxO, 0, g0, sG0)

        _issue_inputs(0, sidxA, didxA, nbA, sIA)
        _wait_inputs(sidxA, didxA, nbA, sIA)
        _offsets(sidxA)
        _g_issue(sidxA, 0, g0, sG0)

        def _sloop(sup, _):
            @pl.when(sup % 2 == 0)
            def _():
                _super(sup, (sidxA, didxA, nbA), (sidxB, didxB, nbB), sIB)
            @pl.when(sup % 2 == 1)
            def _():
                _super(sup, (sidxB, didxB, nbB), (sidxA, didxA, nbA), sIA)
            return _
        lax.fori_loop(0, nsup, _sloop, None)
        _s_wait(g1, sS1)
        plsc.subcore_barrier()

        def _out(k2, _):
            pltpu.sync_copy(acc.at[pl.ds(s * RPT + k2 * 64, 64)], zb)
            pltpu.sync_copy(zb, agg_h.at[c, pl.ds(s * RPT + k2 * 64, 64)])
            return _
        lax.fori_loop(0, RPT // 64, _out, None)

    return _k


_agg1_kernel = _make_agg(16, EPW // (SUBS * 128), True)
_agg64_kernel = _make_agg(32, EPT // (SUBS * 128), False)


@functools.partial(
    pl.kernel, mesh=_mesh,
    compiler_params=pltpu.CompilerParams(
        needs_layout_passes=False, use_tc_tiling_on_sc=False),
    out_type=jax.ShapeDtypeStruct((NC, 24, 64), jnp.float32),
    scratch_types=[
        pltpu.VMEM_SHARED((24, 64), jnp.float32),
        pltpu.VMEM((SUBS, 128), jnp.int32),
        pltpu.VMEM((SUBS, 128), jnp.int32),
        pltpu.VMEM((SUBS, 128), jnp.float32),
        pltpu.VMEM((SUBS, 128), jnp.int32),
        pltpu.VMEM((SUBS, 128), jnp.int32),
        pltpu.VMEM((SUBS, 128), jnp.float32),
        pltpu.VMEM((16, 64), jnp.float32),
        pltpu.VMEM((16,), jnp.int32),
        pltpu.VMEM((16,), jnp.int32),
        pltpu.SemaphoreType.DMA,
        pltpu.SemaphoreType.DMA,
    ],
)
def _agg19_kernel(src_h, dst_h, norm_h, h3_h, agg_h,
                  acc, sidxA, didxA, nbA, sidxB, didxB, nbB,
                  gb, si16, di16, sIA, sIB):
    c = lax.axis_index("c")
    s = lax.axis_index("s")
    wrk = s * NC + c
    nsup = EPW // (SUBS * 128)
    base_rows = wrk * nsup * SUBS

    @pl.when(s == 0)
    def _():
        for i in range(16):
            for q in range(4):
                gb[i, pl.ds(q * 16, 16)] = jnp.zeros((16,), jnp.float32)
        pltpu.sync_copy(gb, acc.at[pl.ds(0, 16)])
        pltpu.sync_copy(gb.at[pl.ds(0, 8)], acc.at[pl.ds(16, 8)])
    plsc.subcore_barrier()

    def _issue_inputs(sup, sidx, didx, nb, sem):
        r0 = base_rows + sup * SUBS
        pltpu.async_copy(src_h.at[pl.ds(r0, SUBS)], sidx, sem)
        pltpu.async_copy(dst_h.at[pl.ds(r0, SUBS)], didx, sem)
        pltpu.async_copy(norm_h.at[pl.ds(r0, SUBS)], nb, sem)

    def _wait_inputs(sidx, didx, nb, sem):
        pltpu.make_async_copy(src_h.at[pl.ds(0, SUBS)], sidx, sem).wait()
        pltpu.make_async_copy(dst_h.at[pl.ds(0, SUBS)], didx, sem).wait()
        pltpu.make_async_copy(norm_h.at[pl.ds(0, SUBS)], nb, sem).wait()

    def _super(sup, P, O, semO):
        sidxP, didxP, nbP = P
        sidxO, didxO, nbO = O
        @pl.when(sup < nsup - 1)
        def _():
            _issue_inputs(sup + 1, sidxO, didxO, nbO, semO)

        def _sub(r, _):
            rowv = jnp.zeros((16,), jnp.int32) + r
            m = didxP[r, pl.ds(0, 16)]
            for g in range(1, 8):
                m = jnp.minimum(m, didxP[r, pl.ds(g * 16, 16)])
            subhit = jnp.min(m) < NUM_GNBS
            @pl.when(subhit)
            def _():
                _sub_slow(r, rowv)
            return _

        def _sub_slow(r, rowv):
            for g in range(8):
                o = g * 16
                d16 = didxP[r, pl.ds(o, 16)]
                hit = jnp.min(d16) < NUM_GNBS
                @pl.when(hit)
                def _():
                    s16 = sidxP[r, pl.ds(o, 16)]
                    n16 = nbP[r, pl.ds(o, 16)]
                    si16[...] = s16
                    di16[...] = jnp.where(d16 < NUM_GNBS, d16, 20)
                    pltpu.sync_copy(h3_h.at[si16], gb)
                    for e in range(16):
                        colv = jnp.zeros((16,), jnp.int32) + (o + e)
                        spl = plsc.load_gather(nbP, [rowv, colv])
                        for q in range(4):
                            qo = q * 16
                            gb[e, pl.ds(qo, 16)] = gb[e, pl.ds(qo, 16)] * spl
                    pltpu.sync_copy(gb, acc.at[di16], add=True)
            return _
        lax.fori_loop(0, SUBS, _sub, None)

        @pl.when(sup < nsup - 1)
        def _():
            _wait_inputs(sidxO, didxO, nbO, semO)

    _issue_inputs(0, sidxA, didxA, nbA, sIA)
    _wait_inputs(sidxA, didxA, nbA, sIA)

    def _sloop(sup, _):
        @pl.when(sup % 2 == 0)
        def _():
            _super(sup, (sidxA, didxA, nbA), (sidxB, didxB, nbB), sIB)
        @pl.when(sup % 2 == 1)
        def _():
            _super(sup, (sidxB, didxB, nbB), (sidxA, didxA, nbA), sIA)
        return _
    lax.fori_loop(0, EPW // (SUBS * 128), _sloop, None)
    plsc.subcore_barrier()

    @pl.when(s == 0)
    def _():
        pltpu.sync_copy(acc.at[pl.ds(0, 16)], gb)
        pltpu.sync_copy(gb, agg_h.at[c, pl.ds(0, 16)])
        pltpu.sync_copy(acc.at[pl.ds(16, 8)], gb.at[pl.ds(0, 8)])
        pltpu.sync_copy(gb.at[pl.ds(0, 8)], agg_h.at[c, pl.ds(16, 8)])


def _mm1_body(agg_ref, w_ref, b_ref, o_ref):
    a = agg_ref[0] + agg_ref[1]
    z = jnp.dot(a, w_ref[...], preferred_element_type=jnp.float32) + b_ref[...]
    h = jnp.where(z >= 0, z, 0.01 * z)
    o_ref[0] = h[:, 0:32]
    o_ref[1] = h[:, 32:64]


def _mm1(aggp, w1p, b1):
    return pl.pallas_call(
        _mm1_body,
        grid=(NP // 512,),
        in_specs=[
            pl.BlockSpec((NC, 512, 16), lambda i: (0, i, 0)),
            pl.BlockSpec((16, 64), lambda i: (0, 0)),
            pl.BlockSpec((1, 64), lambda i: (0, 0)),
        ],
        out_specs=pl.BlockSpec((NC, 512, 32), lambda i: (0, i, 0)),
        out_shape=jax.ShapeDtypeStruct((NC, NP, 32), jnp.float32),
    )(aggp, w1p, b1.reshape(1, 64))


def _mm64_body(agg_ref, w_ref, b_ref, o_ref):
    w = w_ref[...]
    z = (jnp.dot(agg_ref[0], w[0:32, :], preferred_element_type=jnp.float32)
         + jnp.dot(agg_ref[1], w[32:64, :], preferred_element_type=jnp.float32)
         + b_ref[...])
    h = jnp.where(z >= 0, z, 0.01 * z)
    o_ref[0] = h[:, 0:32]
    o_ref[1] = h[:, 32:64]


def _mm64(agg, w, b):
    return pl.pallas_call(
        _mm64_body,
        grid=(NP // 512,),
        in_specs=[
            pl.BlockSpec((NC, 512, 32), lambda i: (0, i, 0)),
            pl.BlockSpec((64, 64), lambda i: (0, 0)),
            pl.BlockSpec((1, 64), lambda i: (0, 0)),
        ],
        out_specs=pl.BlockSpec((NC, 512, 32), lambda i: (0, i, 0)),
        out_shape=jax.ShapeDtypeStruct((NC, NP, 32), jnp.float32),
    )(agg, w, b.reshape(1, 64))


def _mm64f_body(agg_ref, w_ref, b_ref, of_ref):
    w = w_ref[...]
    z = (jnp.dot(agg_ref[0], w[0:32, :], preferred_element_type=jnp.float32)
         + jnp.dot(agg_ref[1], w[32:64, :], preferred_element_type=jnp.float32)
         + b_ref[...])
    of_ref[...] = jnp.where(z >= 0, z, 0.01 * z)


def _mm64f(agg, w, b):
    return pl.pallas_call(
        _mm64f_body,
        grid=(NP // 512,),
        in_specs=[
            pl.BlockSpec((NC, 512, 32), lambda i: (0, i, 0)),
            pl.BlockSpec((64, 64), lambda i: (0, 0)),
            pl.BlockSpec((1, 64), lambda i: (0, 0)),
        ],
        out_specs=pl.BlockSpec((512, 64), lambda i: (i, 0)),
        out_shape=jax.ShapeDtypeStruct((NP, 64), jnp.float32),
    )(agg, w, b.reshape(1, 64))


def _final_body(a_ref, w4_ref, b4_ref, wl_ref, bl_ref, o_ref):
    i = pl.program_id(0)
    o_ref[...] = jnp.full((512, 5), -jnp.inf, jnp.float32)
    @pl.when(i == 0)
    def _():
        a = a_ref[0] + a_ref[1]
        z4 = jnp.dot(a, w4_ref[...], preferred_element_type=jnp.float32) + b4_ref[...]
        h4 = jnp.where(z4 >= 0, z4, 0.01 * z4)
        z = jnp.dot(h4, wl_ref[...], preferred_element_type=jnp.float32) + bl_ref[...]
        rows = lax.broadcasted_iota(jnp.int32, (24, 5), 0)
        o_ref[0:24, :] = jnp.where(rows < NUM_GNBS, z, -jnp.inf)


def _final(agg4p, w4, b4, wl, bl):
    return pl.pallas_call(
        _final_body,
        grid=(98,),
        in_specs=[
            pl.BlockSpec((NC, 24, 64), lambda i: (0, 0, 0)),
            pl.BlockSpec((64, 64), lambda i: (0, 0)),
            pl.BlockSpec((1, 64), lambda i: (0, 0)),
            pl.BlockSpec((64, 5), lambda i: (0, 0)),
            pl.BlockSpec((1, 5), lambda i: (0, 0)),
        ],
        out_specs=pl.BlockSpec((512, 5), lambda i: (i, 0)),
        out_shape=jax.ShapeDtypeStruct((N, 5), jnp.float32),
    )(agg4p, w4, b4.reshape(1, 64), wl, bl.reshape(1, 5))


def kernel(x, edges, weights, W1, b1, W2, b2, W3, b3, W4, b4, Wl, bl):
    npad = EP - (E + N)
    loop = jnp.arange(N, dtype=jnp.int32)
    zpad = jnp.zeros((npad,), jnp.int32)
    src = jnp.concatenate([edges[0], loop, zpad])
    dst = jnp.concatenate([edges[1], loop, zpad])
    ew = jnp.concatenate([weights, jnp.ones((N,), jnp.float32),
                          jnp.zeros((npad,), jnp.float32)])

    x_pad = jnp.zeros((NP, 16), jnp.float32).at[:N, :IN_F].set(x)
    w1p = jnp.zeros((16, H), jnp.float32).at[:IN_F].set(W1)

    src2 = src.reshape(EP // 128, 128)
    dst2 = dst.reshape(EP // 128, 128)
    ew2 = ew.reshape(EP // 128, 128)

    degp = _deg_kernel(dst2, ew2)
    dinv = _dinv(degp)
    norm2 = _norm_kernel(src2, dst2, ew2, dinv)
    aggp = _agg1_kernel(src2, dst2, norm2, x_pad)
    h1 = _mm1(aggp, w1p, b1)
    agg2 = _agg64_kernel(src2, dst2, norm2, h1.reshape(NC * NP, 32))
    h2 = _mm64(agg2, W2, b2)
    agg3 = _agg64_kernel(src2, dst2, norm2, h2.reshape(NC * NP, 32))
    h3full = _mm64f(agg3, W3, b3)
    agg4p = _agg19_kernel(src2, dst2, norm2, h3full)
    return _final(agg4p, W4, b4, Wl, bl)


# speedup vs baseline: 1.2849x; 1.2849x over previous
import functools

import jax
import jax.numpy as jnp
from jax import lax
from jax.experimental import pallas as pl
from jax.experimental.pallas import tpu as pltpu
from jax.experimental.pallas import tpu_sc as plsc

N = 50000
E = 800000
NUM_GNBS = 19
H = 64
IN_F = 6

NP = 50176
EP = 860160
NC = 2
NS = 16
NW = NC * NS
EPW = EP // NW
EPT = EP // NS
CH = 128
RPT = NP // NS

_mesh = plsc.VectorSubcoreMesh(core_axis_name="c", subcore_axis_name="s")


_NSUPW = 15


@functools.partial(
    pl.kernel, mesh=_mesh,
    compiler_params=pltpu.CompilerParams(needs_layout_passes=False, use_tc_tiling_on_sc=False),
    out_type=jax.ShapeDtypeStruct((NC * NP,), jnp.float32),
    scratch_types=[
        pltpu.VMEM_SHARED((NP,), jnp.float32),
        pltpu.VMEM((14, 128), jnp.int32),
        pltpu.VMEM((14, 128), jnp.float32),
        pltpu.VMEM((14, 128), jnp.int32),
        pltpu.VMEM((14, 128), jnp.float32),
        pltpu.VMEM((RPT,), jnp.float32),
        pltpu.SemaphoreType.DMA,
        pltpu.SemaphoreType.DMA,
        pltpu.SemaphoreType.DMA,
        pltpu.SemaphoreType.DMA,
    ],
)
def _deg_kernel(dst_h, ew_h, deg_h, acc, didxA, ewA, didxB, ewB, zb,
                sIA, sIB, sS0, sS1):
    c = lax.axis_index("c")
    s = lax.axis_index("s")
    wrk = s * NC + c
    base_rows = wrk * _NSUPW * SUBS
    def _z(i, _):
        zb[pl.ds(pl.multiple_of(i * 16, 16), 16)] = jnp.zeros((16,), jnp.float32)
        return _
    lax.fori_loop(0, RPT // 16, _z, None)
    pltpu.sync_copy(zb, acc.at[pl.ds(s * RPT, RPT)])
    plsc.subcore_barrier()

    def _issue_inputs(sup, didx, ewb, sem):
        r0 = base_rows + sup * SUBS
        pltpu.async_copy(dst_h.at[pl.ds(r0, SUBS)], didx, sem)
        pltpu.async_copy(ew_h.at[pl.ds(r0, SUBS)], ewb, sem)

    def _wait_inputs(didx, ewb, sem):
        pltpu.make_async_copy(dst_h.at[pl.ds(0, SUBS)], didx, sem).wait()
        pltpu.make_async_copy(ew_h.at[pl.ds(0, SUBS)], ewb, sem).wait()

    def _s_wait(ewb, sem):
        pltpu.make_async_copy(ewb.at[0], acc.at[pl.ds(0, 128)], sem).wait()

    def _super(sup, didxP, ewP, didxO, ewO, semO):
        @pl.when(sup < _NSUPW - 1)
        def _():
            _issue_inputs(sup + 1, didxO, ewO, semO)
        def _sub(r, _):
            @pl.when(r % 2 == 0)
            def _():
                @pl.when(r >= 2)
                def _():
                    _s_wait(ewP, sS0)
                pltpu.async_copy(ewP.at[r], acc.at[didxP.at[r]], sS0, add=True)
            @pl.when(r % 2 == 1)
            def _():
                @pl.when(r >= 2)
                def _():
                    _s_wait(ewP, sS1)
                pltpu.async_copy(ewP.at[r], acc.at[didxP.at[r]], sS1, add=True)
            return _
        lax.fori_loop(0, SUBS, _sub, None)
        _s_wait(ewP, sS0)
        _s_wait(ewP, sS1)
        @pl.when(sup < _NSUPW - 1)
        def _():
            _wait_inputs(didxO, ewO, semO)

    _issue_inputs(0, didxA, ewA, sIA)
    _wait_inputs(didxA, ewA, sIA)
    def _sloop(sup, _):
        @pl.when(sup % 2 == 0)
        def _():
            _super(sup, didxA, ewA, didxB, ewB, sIB)
        @pl.when(sup % 2 == 1)
        def _():
            _super(sup, didxB, ewB, didxA, ewA, sIA)
        return _
    lax.fori_loop(0, _NSUPW, _sloop, None)
    plsc.subcore_barrier()
    pltpu.sync_copy(acc.at[pl.ds(s * RPT, RPT)], zb)
    pltpu.sync_copy(zb, deg_h.at[pl.ds(c * NP + s * RPT, RPT)])


def _dinv_body(deg_ref, o_ref):
    o_ref[...] = lax.rsqrt(deg_ref[0] + deg_ref[1])


def _dinv(degp):
    return pl.pallas_call(
        _dinv_body,
        out_shape=jax.ShapeDtypeStruct((392, 128), jnp.float32),
    )(degp.reshape(NC, 392, 128)).reshape(NP)


@functools.partial(
    pl.kernel, mesh=_mesh,
    compiler_params=pltpu.CompilerParams(needs_layout_passes=False, use_tc_tiling_on_sc=False),
    out_type=jax.ShapeDtypeStruct((EP // 128, 128), jnp.float32),
    scratch_types=[
        pltpu.VMEM((N,), jnp.float32),
        pltpu.VMEM((14, 128), jnp.int32),
        pltpu.VMEM((14, 128), jnp.int32),
        pltpu.VMEM((14, 128), jnp.float32),
        pltpu.VMEM((14, 128), jnp.int32),
        pltpu.VMEM((14, 128), jnp.int32),
        pltpu.VMEM((14, 128), jnp.float32),
        pltpu.VMEM((14, 128), jnp.float32),
        pltpu.VMEM((14, 128), jnp.float32),
        pltpu.SemaphoreType.DMA,
        pltpu.SemaphoreType.DMA,
        pltpu.SemaphoreType.DMA,
        pltpu.SemaphoreType.DMA,
    ],
)
def _norm_kernel(src_h, dst_h, ew_h, dinv_h, norm_h, dinvb,
                 sbA, dbA, ebA, sbB, dbB, ebB, nbA, nbB,
                 sIA, sIB, sOA, sOB):
    c = lax.axis_index("c")
    s = lax.axis_index("s")
    wrk = s * NC + c
    base_rows = wrk * _NSUPW * SUBS
    pltpu.sync_copy(dinv_h.at[pl.ds(0, N)], dinvb)

    def _issue_inputs(sup, sb, db, eb, sem):
        r0 = base_rows + sup * SUBS
        pltpu.async_copy(src_h.at[pl.ds(r0, SUBS)], sb, sem)
        pltpu.async_copy(dst_h.at[pl.ds(r0, SUBS)], db, sem)
        pltpu.async_copy(ew_h.at[pl.ds(r0, SUBS)], eb, sem)

    def _wait_inputs(sb, db, eb, sem):
        pltpu.make_async_copy(src_h.at[pl.ds(0, SUBS)], sb, sem).wait()
        pltpu.make_async_copy(dst_h.at[pl.ds(0, SUBS)], db, sem).wait()
        pltpu.make_async_copy(ew_h.at[pl.ds(0, SUBS)], eb, sem).wait()

    def _o_wait(nb, sem):
        pltpu.make_async_copy(norm_h.at[pl.ds(0, SUBS)], nb, sem).wait()

    def _super(sup, P, O, semO, nbP, semOutP):
        sbP, dbP, ebP = P
        sbO, dbO, ebO = O
        @pl.when(sup < _NSUPW - 1)
        def _():
            _issue_inputs(sup + 1, sbO, dbO, ebO, semO)
        @pl.when(sup >= 2)
        def _():
            _o_wait(nbP, semOutP)
        def _sub(r, _):
            for g in range(8):
                o = g * 16
                s16 = sbP[r, pl.ds(o, 16)]
                d16 = dbP[r, pl.ds(o, 16)]
                e16 = ebP[r, pl.ds(o, 16)]
                dv_s = plsc.load_gather(dinvb, [s16])
                dv_d = plsc.load_gather(dinvb, [d16])
                nbP[r, pl.ds(o, 16)] = dv_s * e16 * dv_d
            return _
        lax.fori_loop(0, SUBS, _sub, None)
        pltpu.async_copy(nbP, norm_h.at[pl.ds(base_rows + sup * SUBS, SUBS)],
                         semOutP)
        @pl.when(sup < _NSUPW - 1)
        def _():
            _wait_inputs(sbO, dbO, ebO, semO)

    _issue_inputs(0, sbA, dbA, ebA, sIA)
    _wait_inputs(sbA, dbA, ebA, sIA)
    def _sloop(sup, _):
        @pl.when(sup % 2 == 0)
        def _():
            _super(sup, (sbA, dbA, ebA), (sbB, dbB, ebB), sIB, nbA, sOA)
        @pl.when(sup % 2 == 1)
        def _():
            _super(sup, (sbB, dbB, ebB), (sbA, dbA, ebA), sIA, nbB, sOB)
        return _
    lax.fori_loop(0, _NSUPW, _sloop, None)
    _o_wait(nbB, sOB)
    _o_wait(nbA, sOA)


SUBS = 14
PAIRS = SUBS // 2


def _make_agg(W, nsup, edge_split):
    @functools.partial(
        pl.kernel, mesh=_mesh,
        compiler_params=pltpu.CompilerParams(
            needs_layout_passes=False, use_tc_tiling_on_sc=False),
        out_type=jax.ShapeDtypeStruct((NC, NP, W), jnp.float32),
        scratch_types=[
            pltpu.VMEM_SHARED((NP, W), jnp.float32),
            pltpu.VMEM((SUBS, 128), jnp.int32),
            pltpu.VMEM((SUBS, 128), jnp.int32),
            pltpu.VMEM((SUBS, 128), jnp.float32),
            pltpu.VMEM((SUBS, 128), jnp.int32),
            pltpu.VMEM((SUBS, 128), jnp.int32),
            pltpu.VMEM((SUBS, 128), jnp.float32),
            pltpu.VMEM((128, W), jnp.float32),
            pltpu.VMEM((128, W), jnp.float32),
            pltpu.VMEM((64, W), jnp.float32),
            pltpu.SemaphoreType.DMA,
            pltpu.SemaphoreType.DMA,
            pltpu.SemaphoreType.DMA,
            pltpu.SemaphoreType.DMA,
            pltpu.SemaphoreType.DMA,
            pltpu.SemaphoreType.DMA,
        ],
    )
    def _k(src_h, dst_h, norm_h, tab_h, agg_h,
           acc, sidxA, didxA, nbA, sidxB, didxB, nbB, g0, g1, zb,
           sIA, sIB, sG0, sG1, sS0, sS1):
        c = lax.axis_index("c")
        s = lax.axis_index("s")
        if edge_split:
            wrk = s * NC + c
        else:
            wrk = s
        base_rows = wrk * nsup * SUBS
        roffv = jnp.zeros((16,), jnp.int32) + c * NP

        for i in range(64):
            for q in range(W // 16):
                zb[i, pl.ds(q * 16, 16)] = jnp.zeros((16,), jnp.float32)
        def _zl(k2, _):
            pltpu.sync_copy(zb, acc.at[pl.ds(s * RPT + k2 * 64, 64)])
            return _
        lax.fori_loop(0, RPT // 64, _zl, None)
        plsc.subcore_barrier()

        def _issue_inputs(sup, sidx, didx, nb, sem):
            r0 = base_rows + sup * SUBS
            pltpu.async_copy(src_h.at[pl.ds(r0, SUBS)], sidx, sem)
            pltpu.async_copy(dst_h.at[pl.ds(r0, SUBS)], didx, sem)
            pltpu.async_copy(norm_h.at[pl.ds(r0, SUBS)], nb, sem)

        def _wait_inputs(sidx, didx, nb, sem):
            pltpu.make_async_copy(src_h.at[pl.ds(0, SUBS)], sidx, sem).wait()
            pltpu.make_async_copy(dst_h.at[pl.ds(0, SUBS)], didx, sem).wait()
            pltpu.make_async_copy(norm_h.at[pl.ds(0, SUBS)], nb, sem).wait()

        def _offsets(sidx):
            if edge_split:
                return
            def _orow(r, _):
                def _og(g, _2):
                    o = pl.multiple_of(g * 16, 16)
                    sidx[r, pl.ds(o, 16)] = sidx[r, pl.ds(o, 16)] + roffv
                    return _2
                lax.fori_loop(0, 8, _og, None)
                return _
            lax.fori_loop(0, SUBS, _orow, None)

        def _g_issue(sidx, row, gb, sem):
            pltpu.async_copy(tab_h.at[sidx.at[row]], gb, sem)

        def _g_wait(gb, sem):
            pltpu.make_async_copy(tab_h.at[pl.ds(0, 128)], gb, sem).wait()

        def _s_issue(didx, row, gb, sem):
            pltpu.async_copy(gb, acc.at[didx.at[row]], sem, add=True)

        def _s_wait(gb, sem):
            pltpu.make_async_copy(gb, acc.at[pl.ds(0, 128)], sem).wait()

        def _scale(gb, nb, rowv):
            def _se(e, _):
                colv = jnp.zeros((16,), jnp.int32) + e
                spl = plsc.load_gather(nb, [rowv, colv])
                for q in range(W // 16):
                    o = q * 16
                    gb[e, pl.ds(o, 16)] = gb[e, pl.ds(o, 16)] * spl
                return _
            lax.fori_loop(0, 128, _se, None, unroll=16)

        def _super(sup, P, O, semO):
            sidxP, didxP, nbP = P
            sidxO, didxO, nbO = O
            @pl.when(sup > 0)
            def _():
                _s_wait(g1, sS1)
            @pl.when(sup < nsup - 1)
            def _():
                _issue_inputs(sup + 1, sidxO, didxO, nbO, semO)

            def _pair(t, _):
                a2 = 2 * t
                rowa = jnp.zeros((16,), jnp.int32) + a2
                _g_wait(g0, sG0)
                @pl.when(t > 0)
                def _():
                    _s_wait(g1, sS1)
                _g_issue(sidxP, a2 + 1, g1, sG1)
                _scale(g0, nbP, rowa)
                _s_issue(didxP, a2, g0, sS0)
                _g_wait(g1, sG1)
                _scale(g1, nbP, rowa + 1)
                _s_wait(g0, sS0)
                @pl.when(t < PAIRS - 1)
                def _():
                    _g_issue(sidxP, a2 + 2, g0, sG0)
                _s_issue(didxP, a2 + 1, g1, sS1)
                return _
            lax.fori_loop(0, PAIRS, _pair, None)

            @pl.when(sup < nsup - 1)
            def _():
                _wait_inputs(sidxO, didxO, nbO, semO)
                _offsets(sidxO)
                _g_issue(sidxO, 0, g0, sG0)

        _issue_inputs(0, sidxA, didxA, nbA, sIA)
        _wait_inputs(sidxA, didxA, nbA, sIA)
        _offsets(sidxA)
        _g_issue(sidxA, 0, g0, sG0)

        def _sloop(sup, _):
            @pl.when(sup % 2 == 0)
            def _():
                _super(sup, (sidxA, didxA, nbA), (sidxB, didxB, nbB), sIB)
            @pl.when(sup % 2 == 1)
            def _():
                _super(sup, (sidxB, didxB, nbB), (sidxA, didxA, nbA), sIA)
            return _
        lax.fori_loop(0, nsup, _sloop, None)
        _s_wait(g1, sS1)
        plsc.subcore_barrier()

        def _out(k2, _):
            pltpu.sync_copy(acc.at[pl.ds(s * RPT + k2 * 64, 64)], zb)
            pltpu.sync_copy(zb, agg_h.at[c, pl.ds(s * RPT + k2 * 64, 64)])
            return _
        lax.fori_loop(0, RPT // 64, _out, None)

    return _k


_agg1_kernel = _make_agg(16, EPW // (SUBS * 128), True)
_agg64_kernel = _make_agg(32, EPT // (SUBS * 128), False)


@functools.partial(
    pl.kernel, mesh=_mesh,
    compiler_params=pltpu.CompilerParams(
        needs_layout_passes=False, use_tc_tiling_on_sc=False),
    out_type=jax.ShapeDtypeStruct((NC, 24, 64), jnp.float32),
    scratch_types=[
        pltpu.VMEM_SHARED((24, 64), jnp.float32),
        pltpu.VMEM((SUBS, 128), jnp.int32),
        pltpu.VMEM((SUBS, 128), jnp.int32),
        pltpu.VMEM((SUBS, 128), jnp.float32),
        pltpu.VMEM((SUBS, 128), jnp.int32),
        pltpu.VMEM((SUBS, 128), jnp.int32),
        pltpu.VMEM((SUBS, 128), jnp.float32),
        pltpu.VMEM((16, 64), jnp.float32),
        pltpu.VMEM((16,), jnp.int32),
        pltpu.VMEM((16,), jnp.int32),
        pltpu.SemaphoreType.DMA,
        pltpu.SemaphoreType.DMA,
    ],
)
def _agg19_kernel(src_h, dst_h, norm_h, h3_h, agg_h,
                  acc, sidxA, didxA, nbA, sidxB, didxB, nbB,
                  gb, si16, di16, sIA, sIB):
    c = lax.axis_index("c")
    s = lax.axis_index("s")
    wrk = s * NC + c
    nsup = EPW // (SUBS * 128)
    base_rows = wrk * nsup * SUBS

    @pl.when(s == 0)
    def _():
        for i in range(16):
            for q in range(4):
                gb[i, pl.ds(q * 16, 16)] = jnp.zeros((16,), jnp.float32)
        pltpu.sync_copy(gb, acc.at[pl.ds(0, 16)])
        pltpu.sync_copy(gb.at[pl.ds(0, 8)], acc.at[pl.ds(16, 8)])
    plsc.subcore_barrier()

    def _issue_inputs(sup, sidx, didx, nb, sem):
        r0 = base_rows + sup * SUBS
        pltpu.async_copy(src_h.at[pl.ds(r0, SUBS)], sidx, sem)
        pltpu.async_copy(dst_h.at[pl.ds(r0, SUBS)], didx, sem)
        pltpu.async_copy(norm_h.at[pl.ds(r0, SUBS)], nb, sem)

    def _wait_inputs(sidx, didx, nb, sem):
        pltpu.make_async_copy(src_h.at[pl.ds(0, SUBS)], sidx, sem).wait()
        pltpu.make_async_copy(dst_h.at[pl.ds(0, SUBS)], didx, sem).wait()
        pltpu.make_async_copy(norm_h.at[pl.ds(0, SUBS)], nb, sem).wait()

    def _super(sup, P, O, semO):
        sidxP, didxP, nbP = P
        sidxO, didxO, nbO = O
        @pl.when(sup < nsup - 1)
        def _():
            _issue_inputs(sup + 1, sidxO, didxO, nbO, semO)

        def _sub(r, _):
            rowv = jnp.zeros((16,), jnp.int32) + r
            m = didxP[r, pl.ds(0, 16)]
            for g in range(1, 8):
                m = jnp.minimum(m, didxP[r, pl.ds(g * 16, 16)])
            subhit = jnp.min(m) < NUM_GNBS
            @pl.when(subhit)
            def _():
                _sub_slow(r, rowv)
            return _

        def _sub_slow(r, rowv):
            for g in range(8):
                o = g * 16
                d16 = didxP[r, pl.ds(o, 16)]
                hit = jnp.min(d16) < NUM_GNBS
                @pl.when(hit)
                def _():
                    s16 = sidxP[r, pl.ds(o, 16)]
                    n16 = nbP[r, pl.ds(o, 16)]
                    si16[...] = s16
                    di16[...] = jnp.where(d16 < NUM_GNBS, d16, 20)
                    pltpu.sync_copy(h3_h.at[si16], gb)
                    for e in range(16):
                        colv = jnp.zeros((16,), jnp.int32) + (o + e)
                        spl = plsc.load_gather(nbP, [rowv, colv])
                        for q in range(4):
                            qo = q * 16
                            gb[e, pl.ds(qo, 16)] = gb[e, pl.ds(qo, 16)] * spl
                    pltpu.sync_copy(gb, acc.at[di16], add=True)
            return _
        lax.fori_loop(0, SUBS, _sub, None)

        @pl.when(sup < nsup - 1)
        def _():
            _wait_inputs(sidxO, didxO, nbO, semO)

    _issue_inputs(0, sidxA, didxA, nbA, sIA)
    _wait_inputs(sidxA, didxA, nbA, sIA)

    def _sloop(sup, _):
        @pl.when(sup % 2 == 0)
        def _():
            _super(sup, (sidxA, didxA, nbA), (sidxB, didxB, nbB), sIB)
        @pl.when(sup % 2 == 1)
        def _():
            _super(sup, (sidxB, didxB, nbB), (sidxA, didxA, nbA), sIA)
        return _
    lax.fori_loop(0, EPW // (SUBS * 128), _sloop, None)
    plsc.subcore_barrier()

    @pl.when(s == 0)
    def _():
        pltpu.sync_copy(acc.at[pl.ds(0, 16)], gb)
        pltpu.sync_copy(gb, agg_h.at[c, pl.ds(0, 16)])
        pltpu.sync_copy(acc.at[pl.ds(16, 8)], gb.at[pl.ds(0, 8)])
        pltpu.sync_copy(gb.at[pl.ds(0, 8)], agg_h.at[c, pl.ds(16, 8)])


def _mm1_body(agg_ref, w_ref, b_ref, o_ref):
    a = agg_ref[0] + agg_ref[1]
    z = jnp.dot(a, w_ref[...], preferred_element_type=jnp.float32) + b_ref[...]
    h = jnp.where(z >= 0, z, 0.01 * z)
    o_ref[0] = h[:, 0:32]
    o_ref[1] = h[:, 32:64]


def _mm1(aggp, w1p, b1):
    return pl.pallas_call(
        _mm1_body,
        grid=(NP // 512,),
        in_specs=[
            pl.BlockSpec((NC, 512, 16), lambda i: (0, i, 0)),
            pl.BlockSpec((16, 64), lambda i: (0, 0)),
            pl.BlockSpec((1, 64), lambda i: (0, 0)),
        ],
        out_specs=pl.BlockSpec((NC, 512, 32), lambda i: (0, i, 0)),
        out_shape=jax.ShapeDtypeStruct((NC, NP, 32), jnp.float32),
    )(aggp, w1p, b1.reshape(1, 64))


def _mm64_body(agg_ref, w_ref, b_ref, o_ref):
    w = w_ref[...]
    z = (jnp.dot(agg_ref[0], w[0:32, :], preferred_element_type=jnp.float32)
         + jnp.dot(agg_ref[1], w[32:64, :], preferred_element_type=jnp.float32)
         + b_ref[...])
    h = jnp.where(z >= 0, z, 0.01 * z)
    o_ref[0] = h[:, 0:32]
    o_ref[1] = h[:, 32:64]


def _mm64(agg, w, b):
    return pl.pallas_call(
        _mm64_body,
        grid=(NP // 512,),
        in_specs=[
            pl.BlockSpec((NC, 512, 32), lambda i: (0, i, 0)),
            pl.BlockSpec((64, 64), lambda i: (0, 0)),
            pl.BlockSpec((1, 64), lambda i: (0, 0)),
        ],
        out_specs=pl.BlockSpec((NC, 512, 32), lambda i: (0, i, 0)),
        out_shape=jax.ShapeDtypeStruct((NC, NP, 32), jnp.float32),
    )(agg, w, b.reshape(1, 64))


def _mm64f_body(agg_ref, w_ref, b_ref, of_ref):
    w = w_ref[...]
    z = (jnp.dot(agg_ref[0], w[0:32, :], preferred_element_type=jnp.float32)
         + jnp.dot(agg_ref[1], w[32:64, :], preferred_element_type=jnp.float32)
         + b_ref[...])
    of_ref[...] = jnp.where(z >= 0, z, 0.01 * z)


def _mm64f(agg, w, b):
    return pl.pallas_call(
        _mm64f_body,
        grid=(NP // 512,),
        in_specs=[
            pl.BlockSpec((NC, 512, 32), lambda i: (0, i, 0)),
            pl.BlockSpec((64, 64), lambda i: (0, 0)),
            pl.BlockSpec((1, 64), lambda i: (0, 0)),
        ],
        out_specs=pl.BlockSpec((512, 64), lambda i: (i, 0)),
        out_shape=jax.ShapeDtypeStruct((NP, 64), jnp.float32),
    )(agg, w, b.reshape(1, 64))


def _final_body(a_ref, w4_ref, b4_ref, wl_ref, bl_ref, o_ref):
    i = pl.program_id(0)
    o_ref[...] = jnp.full((512, 5), -jnp.inf, jnp.float32)
    @pl.when(i == 0)
    def _():
        a = a_ref[0] + a_ref[1]
        z4 = jnp.dot(a, w4_ref[...], preferred_element_type=jnp.float32) + b4_ref[...]
        h4 = jnp.where(z4 >= 0, z4, 0.01 * z4)
        z = jnp.dot(h4, wl_ref[...], preferred_element_type=jnp.float32) + bl_ref[...]
        rows = lax.broadcasted_iota(jnp.int32, (24, 5), 0)
        o_ref[0:24, :] = jnp.where(rows < NUM_GNBS, z, -jnp.inf)


def _final(agg4p, w4, b4, wl, bl):
    return pl.pallas_call(
        _final_body,
        grid=(98,),
        in_specs=[
            pl.BlockSpec((NC, 24, 64), lambda i: (0, 0, 0)),
            pl.BlockSpec((64, 64), lambda i: (0, 0)),
            pl.BlockSpec((1, 64), lambda i: (0, 0)),
            pl.BlockSpec((64, 5), lambda i: (0, 0)),
            pl.BlockSpec((1, 5), lambda i: (0, 0)),
        ],
        out_specs=pl.BlockSpec((512, 5), lambda i: (i, 0)),
        out_shape=jax.ShapeDtypeStruct((N, 5), jnp.float32),
    )(agg4p, w4, b4.reshape(1, 64), wl, bl.reshape(1, 5))


def kernel(x, edges, weights, W1, b1, W2, b2, W3, b3, W4, b4, Wl, bl):
    npad = EP - (E + N)
    loop = jnp.arange(N, dtype=jnp.int32)
    zpad = jnp.zeros((npad,), jnp.int32)
    src = jnp.concatenate([edges[0], loop, zpad])
    dst = jnp.concatenate([edges[1], loop, jnp.full((npad,), N - 1, jnp.int32)])
    ew = jnp.concatenate([weights, jnp.ones((N,), jnp.float32),
                          jnp.zeros((npad,), jnp.float32)])

    x_pad = jnp.zeros((NP, 16), jnp.float32).at[:N, :IN_F].set(x)
    w1p = jnp.zeros((16, H), jnp.float32).at[:IN_F].set(W1)

    src2 = src.reshape(EP // 128, 128)
    dst2 = dst.reshape(EP // 128, 128)
    ew2 = ew.reshape(EP // 128, 128)

    degp = _deg_kernel(dst2, ew2)
    dinv = _dinv(degp)
    norm2 = _norm_kernel(src2, dst2, ew2, dinv)
    aggp = _agg1_kernel(src2, dst2, norm2, x_pad)
    h1 = _mm1(aggp, w1p, b1)
    agg2 = _agg64_kernel(src2, dst2, norm2, h1.reshape(NC * NP, 32))
    h2 = _mm64(agg2, W2, b2)
    agg3 = _agg64_kernel(src2, dst2, norm2, h2.reshape(NC * NP, 32))
    h3full = _mm64f(agg3, W3, b3)
    agg4p = _agg19_kernel(src2, dst2, norm2, h3full)
    return _final(agg4p, W4, b4, Wl, bl)


# speedup vs baseline: 1.3160x; 1.0242x over previous
import functools

import jax
import jax.numpy as jnp
from jax import lax
from jax.experimental import pallas as pl
from jax.experimental.pallas import tpu as pltpu
from jax.experimental.pallas import tpu_sc as plsc

N = 50000
E = 800000
NUM_GNBS = 19
H = 64
IN_F = 6

NP = 50176
EP = 860160
NC = 2
NS = 16
NW = NC * NS
EPW = EP // NW
EPT = EP // NS
CH = 128
RPT = NP // NS

_mesh = plsc.VectorSubcoreMesh(core_axis_name="c", subcore_axis_name="s")


_NSUPW = 15


@functools.partial(
    pl.kernel, mesh=_mesh,
    compiler_params=pltpu.CompilerParams(needs_layout_passes=False, use_tc_tiling_on_sc=False),
    out_type=jax.ShapeDtypeStruct((NC * NP,), jnp.float32),
    scratch_types=[
        pltpu.VMEM_SHARED((NP,), jnp.float32),
        pltpu.VMEM((14, 128), jnp.int32),
        pltpu.VMEM((14, 128), jnp.float32),
        pltpu.VMEM((14, 128), jnp.int32),
        pltpu.VMEM((14, 128), jnp.float32),
        pltpu.VMEM((RPT,), jnp.float32),
        pltpu.SemaphoreType.DMA,
        pltpu.SemaphoreType.DMA,
        pltpu.SemaphoreType.DMA,
        pltpu.SemaphoreType.DMA,
    ],
)
def _deg_kernel(dst_h, ew_h, deg_h, acc, didxA, ewA, didxB, ewB, zb,
                sIA, sIB, sS0, sS1):
    c = lax.axis_index("c")
    s = lax.axis_index("s")
    wrk = s * NC + c
    base_rows = wrk * _NSUPW * SUBS
    def _z(i, _):
        zb[pl.ds(pl.multiple_of(i * 16, 16), 16)] = jnp.zeros((16,), jnp.float32)
        return _
    lax.fori_loop(0, RPT // 16, _z, None)
    pltpu.sync_copy(zb, acc.at[pl.ds(s * RPT, RPT)])
    plsc.subcore_barrier()

    def _issue_inputs(sup, didx, ewb, sem):
        r0 = base_rows + sup * SUBS
        pltpu.async_copy(dst_h.at[pl.ds(r0, SUBS)], didx, sem)
        pltpu.async_copy(ew_h.at[pl.ds(r0, SUBS)], ewb, sem)

    def _wait_inputs(didx, ewb, sem):
        pltpu.make_async_copy(dst_h.at[pl.ds(0, SUBS)], didx, sem).wait()
        pltpu.make_async_copy(ew_h.at[pl.ds(0, SUBS)], ewb, sem).wait()

    def _s_wait(ewb, sem):
        pltpu.make_async_copy(ewb.at[0], acc.at[pl.ds(0, 128)], sem).wait()

    def _super(sup, didxP, ewP, didxO, ewO, semO):
        @pl.when(sup < _NSUPW - 1)
        def _():
            _issue_inputs(sup + 1, didxO, ewO, semO)
        def _sub(r, _):
            @pl.when(r % 2 == 0)
            def _():
                @pl.when(r >= 2)
                def _():
                    _s_wait(ewP, sS0)
                pltpu.async_copy(ewP.at[r], acc.at[didxP.at[r]], sS0, add=True)
            @pl.when(r % 2 == 1)
            def _():
                @pl.when(r >= 2)
                def _():
                    _s_wait(ewP, sS1)
                pltpu.async_copy(ewP.at[r], acc.at[didxP.at[r]], sS1, add=True)
            return _
        lax.fori_loop(0, SUBS, _sub, None)
        _s_wait(ewP, sS0)
        _s_wait(ewP, sS1)
        @pl.when(sup < _NSUPW - 1)
        def _():
            _wait_inputs(didxO, ewO, semO)

    _issue_inputs(0, didxA, ewA, sIA)
    _wait_inputs(didxA, ewA, sIA)
    def _sloop(sup, _):
        @pl.when(sup % 2 == 0)
        def _():
            _super(sup, didxA, ewA, didxB, ewB, sIB)
        @pl.when(sup % 2 == 1)
        def _():
            _super(sup, didxB, ewB, didxA, ewA, sIA)
        return _
    lax.fori_loop(0, _NSUPW, _sloop, None)
    plsc.subcore_barrier()
    pltpu.sync_copy(acc.at[pl.ds(s * RPT, RPT)], zb)
    pltpu.sync_copy(zb, deg_h.at[pl.ds(c * NP + s * RPT, RPT)])


def _dinv_body(deg_ref, o_ref):
    o_ref[...] = lax.rsqrt(deg_ref[0] + deg_ref[1])


def _dinv(degp):
    return pl.pallas_call(
        _dinv_body,
        out_shape=jax.ShapeDtypeStruct((392, 128), jnp.float32),
    )(degp.reshape(NC, 392, 128)).reshape(NP)


@functools.partial(
    pl.kernel, mesh=_mesh,
    compiler_params=pltpu.CompilerParams(needs_layout_passes=False, use_tc_tiling_on_sc=False),
    out_type=jax.ShapeDtypeStruct((EP // 128, 128), jnp.float32),
    scratch_types=[
        pltpu.VMEM((N,), jnp.float32),
        pltpu.VMEM((14, 128), jnp.int32),
        pltpu.VMEM((14, 128), jnp.int32),
        pltpu.VMEM((14, 128), jnp.float32),
        pltpu.VMEM((14, 128), jnp.int32),
        pltpu.VMEM((14, 128), jnp.int32),
        pltpu.VMEM((14, 128), jnp.float32),
        pltpu.VMEM((14, 128), jnp.float32),
        pltpu.VMEM((14, 128), jnp.float32),
        pltpu.SemaphoreType.DMA,
        pltpu.SemaphoreType.DMA,
        pltpu.SemaphoreType.DMA,
        pltpu.SemaphoreType.DMA,
    ],
)
def _norm_kernel(src_h, dst_h, ew_h, dinv_h, norm_h, dinvb,
                 sbA, dbA, ebA, sbB, dbB, ebB, nbA, nbB,
                 sIA, sIB, sOA, sOB):
    c = lax.axis_index("c")
    s = lax.axis_index("s")
    wrk = s * NC + c
    base_rows = wrk * _NSUPW * SUBS
    pltpu.sync_copy(dinv_h.at[pl.ds(0, N)], dinvb)

    def _issue_inputs(sup, sb, db, eb, sem):
        r0 = base_rows + sup * SUBS
        pltpu.async_copy(src_h.at[pl.ds(r0, SUBS)], sb, sem)
        pltpu.async_copy(dst_h.at[pl.ds(r0, SUBS)], db, sem)
        pltpu.async_copy(ew_h.at[pl.ds(r0, SUBS)], eb, sem)

    def _wait_inputs(sb, db, eb, sem):
        pltpu.make_async_copy(src_h.at[pl.ds(0, SUBS)], sb, sem).wait()
        pltpu.make_async_copy(dst_h.at[pl.ds(0, SUBS)], db, sem).wait()
        pltpu.make_async_copy(ew_h.at[pl.ds(0, SUBS)], eb, sem).wait()

    def _o_wait(nb, sem):
        pltpu.make_async_copy(norm_h.at[pl.ds(0, SUBS)], nb, sem).wait()

    def _super(sup, P, O, semO, nbP, semOutP):
        sbP, dbP, ebP = P
        sbO, dbO, ebO = O
        @pl.when(sup < _NSUPW - 1)
        def _():
            _issue_inputs(sup + 1, sbO, dbO, ebO, semO)
        @pl.when(sup >= 2)
        def _():
            _o_wait(nbP, semOutP)
        def _sub(r, _):
            for g in range(8):
                o = g * 16
                s16 = sbP[r, pl.ds(o, 16)]
                d16 = dbP[r, pl.ds(o, 16)]
                e16 = ebP[r, pl.ds(o, 16)]
                dv_s = plsc.load_gather(dinvb, [s16])
                dv_d = plsc.load_gather(dinvb, [d16])
                nbP[r, pl.ds(o, 16)] = dv_s * e16 * dv_d
            return _
        lax.fori_loop(0, SUBS, _sub, None)
        pltpu.async_copy(nbP, norm_h.at[pl.ds(base_rows + sup * SUBS, SUBS)],
                         semOutP)
        @pl.when(sup < _NSUPW - 1)
        def _():
            _wait_inputs(sbO, dbO, ebO, semO)

    _issue_inputs(0, sbA, dbA, ebA, sIA)
    _wait_inputs(sbA, dbA, ebA, sIA)
    def _sloop(sup, _):
        @pl.when(sup % 2 == 0)
        def _():
            _super(sup, (sbA, dbA, ebA), (sbB, dbB, ebB), sIB, nbA, sOA)
        @pl.when(sup % 2 == 1)
        def _():
            _super(sup, (sbB, dbB, ebB), (sbA, dbA, ebA), sIA, nbB, sOB)
        return _
    lax.fori_loop(0, _NSUPW, _sloop, None)
    _o_wait(nbB, sOB)
    _o_wait(nbA, sOA)


SUBS = 14
PAIRS = SUBS // 2


def _make_agg(W, nsup, edge_split):
    @functools.partial(
        pl.kernel, mesh=_mesh,
        compiler_params=pltpu.CompilerParams(
            needs_layout_passes=False, use_tc_tiling_on_sc=False),
        out_type=jax.ShapeDtypeStruct((NC, NP, W), jnp.float32),
        scratch_types=[
            pltpu.VMEM_SHARED((NP, W), jnp.float32),
            pltpu.VMEM((SUBS, 128), jnp.int32),
            pltpu.VMEM((SUBS, 128), jnp.int32),
            pltpu.VMEM((SUBS, 128), jnp.float32),
            pltpu.VMEM((SUBS, 128), jnp.int32),
            pltpu.VMEM((SUBS, 128), jnp.int32),
            pltpu.VMEM((SUBS, 128), jnp.float32),
            pltpu.VMEM((128, W), jnp.float32),
            pltpu.VMEM((128, W), jnp.float32),
            pltpu.VMEM((64, W), jnp.float32),
            pltpu.SemaphoreType.DMA,
            pltpu.SemaphoreType.DMA,
            pltpu.SemaphoreType.DMA,
            pltpu.SemaphoreType.DMA,
            pltpu.SemaphoreType.DMA,
            pltpu.SemaphoreType.DMA,
        ],
    )
    def _k(src_h, dst_h, norm_h, tab_h, agg_h,
           acc, sidxA, didxA, nbA, sidxB, didxB, nbB, g0, g1, zb,
           sIA, sIB, sG0, sG1, sS0, sS1):
        c = lax.axis_index("c")
        s = lax.axis_index("s")
        if edge_split:
            wrk = s * NC + c
        else:
            wrk = s
        base_rows = wrk * nsup * SUBS
        roffv = jnp.zeros((16,), jnp.int32) + c * NP

        for i in range(64):
            for q in range(W // 16):
                zb[i, pl.ds(q * 16, 16)] = jnp.zeros((16,), jnp.float32)
        def _zl(k2, _):
            pltpu.sync_copy(zb, acc.at[pl.ds(s * RPT + k2 * 64, 64)])
            return _
        lax.fori_loop(0, RPT // 64, _zl, None)
        plsc.subcore_barrier()

        def _issue_inputs(sup, sidx, didx, nb, sem):
            r0 = base_rows + sup * SUBS
            pltpu.async_copy(src_h.at[pl.ds(r0, SUBS)], sidx, sem)
            pltpu.async_copy(dst_h.at[pl.ds(r0, SUBS)], didx, sem)
            pltpu.async_copy(norm_h.at[pl.ds(r0, SUBS)], nb, sem)

        def _wait_inputs(sidx, didx, nb, sem):
            pltpu.make_async_copy(src_h.at[pl.ds(0, SUBS)], sidx, sem).wait()
            pltpu.make_async_copy(dst_h.at[pl.ds(0, SUBS)], didx, sem).wait()
            pltpu.make_async_copy(norm_h.at[pl.ds(0, SUBS)], nb, sem).wait()

        def _offsets(sidx):
            if edge_split:
                return
            def _orow(r, _):
                def _og(g, _2):
                    o = pl.multiple_of(g * 16, 16)
                    sidx[r, pl.ds(o, 16)] = sidx[r, pl.ds(o, 16)] + roffv
                    return _2
                lax.fori_loop(0, 8, _og, None)
                return _
            lax.fori_loop(0, SUBS, _orow, None)

        def _g_issue(sidx, row, gb, sem):
            pltpu.async_copy(tab_h.at[sidx.at[row]], gb, sem)

        def _g_wait(gb, sem):
            pltpu.make_async_copy(tab_h.at[pl.ds(0, 128)], gb, sem).wait()

        def _s_issue(didx, row, gb, sem):
            pltpu.async_copy(gb, acc.at[didx.at[row]], sem, add=True)

        def _s_wait(gb, sem):
            pltpu.make_async_copy(gb, acc.at[pl.ds(0, 128)], sem).wait()

        def _scale(gb, nb, rowv):
            def _se(e, _):
                colv = jnp.zeros((16,), jnp.int32) + e
                spl = plsc.load_gather(nb, [rowv, colv])
                for q in range(W // 16):
                    o = q * 16
                    gb[e, pl.ds(o, 16)] = gb[e, pl.ds(o, 16)] * spl
                return _
            lax.fori_loop(0, 128, _se, None, unroll=16)

        def _super(sup, P, O, semO):
            sidxP, didxP, nbP = P
            sidxO, didxO, nbO = O
            @pl.when(sup > 0)
            def _():
                _s_wait(g1, sS1)
            @pl.when(sup < nsup - 1)
            def _():
                _issue_inputs(sup + 1, sidxO, didxO, nbO, semO)

            def _pair(t, _):
                a2 = 2 * t
                rowa = jnp.zeros((16,), jnp.int32) + a2
                @pl.when(t > 0)
                def _():
                    _s_wait(g1, sS1)
                _g_issue(sidxP, a2 + 1, g1, sG1)
                _g_wait(g0, sG0)
                _scale(g0, nbP, rowa)
                _s_issue(didxP, a2, g0, sS0)
                _g_wait(g1, sG1)
                _scale(g1, nbP, rowa + 1)
                _s_wait(g0, sS0)
                @pl.when(t < PAIRS - 1)
                def _():
                    _g_issue(sidxP, a2 + 2, g0, sG0)
                _s_issue(didxP, a2 + 1, g1, sS1)
                return _
            lax.fori_loop(0, PAIRS, _pair, None)

            @pl.when(sup < nsup - 1)
            def _():
                _wait_inputs(sidxO, didxO, nbO, semO)
                _offsets(sidxO)
                _g_issue(sidxO, 0, g0, sG0)

        _issue_inputs(0, sidxA, didxA, nbA, sIA)
        _wait_inputs(sidxA, didxA, nbA, sIA)
        _offsets(sidxA)
        _g_issue(sidxA, 0, g0, sG0)

        def _sloop(sup, _):
            @pl.when(sup % 2 == 0)
            def _():
                _super(sup, (sidxA, didxA, nbA), (sidxB, didxB, nbB), sIB)
            @pl.when(sup % 2 == 1)
            def _():
                _super(sup, (sidxB, didxB, nbB), (sidxA, didxA, nbA), sIA)
            return _
        lax.fori_loop(0, nsup, _sloop, None)
        _s_wait(g1, sS1)
        plsc.subcore_barrier()

        def _out(k2, _):
            pltpu.sync_copy(acc.at[pl.ds(s * RPT + k2 * 64, 64)], zb)
            pltpu.sync_copy(zb, agg_h.at[c, pl.ds(s * RPT + k2 * 64, 64)])
            return _
        lax.fori_loop(0, RPT // 64, _out, None)

    return _k


_agg1_kernel = _make_agg(16, EPW // (SUBS * 128), True)
_agg64_kernel = _make_agg(32, EPT // (SUBS * 128), False)


@functools.partial(
    pl.kernel, mesh=_mesh,
    compiler_params=pltpu.CompilerParams(
        needs_layout_passes=False, use_tc_tiling_on_sc=False),
    out_type=jax.ShapeDtypeStruct((NC, 24, 64), jnp.float32),
    scratch_types=[
        pltpu.VMEM_SHARED((24, 64), jnp.float32),
        pltpu.VMEM((SUBS, 128), jnp.int32),
        pltpu.VMEM((SUBS, 128), jnp.int32),
        pltpu.VMEM((SUBS, 128), jnp.float32),
        pltpu.VMEM((SUBS, 128), jnp.int32),
        pltpu.VMEM((SUBS, 128), jnp.int32),
        pltpu.VMEM((SUBS, 128), jnp.float32),
        pltpu.VMEM((16, 64), jnp.float32),
        pltpu.VMEM((16,), jnp.int32),
        pltpu.VMEM((16,), jnp.int32),
        pltpu.SemaphoreType.DMA,
        pltpu.SemaphoreType.DMA,
    ],
)
def _agg19_kernel(src_h, dst_h, norm_h, h3_h, agg_h,
                  acc, sidxA, didxA, nbA, sidxB, didxB, nbB,
                  gb, si16, di16, sIA, sIB):
    c = lax.axis_index("c")
    s = lax.axis_index("s")
    wrk = s * NC + c
    nsup = EPW // (SUBS * 128)
    base_rows = wrk * nsup * SUBS

    @pl.when(s == 0)
    def _():
        for i in range(16):
            for q in range(4):
                gb[i, pl.ds(q * 16, 16)] = jnp.zeros((16,), jnp.float32)
        pltpu.sync_copy(gb, acc.at[pl.ds(0, 16)])
        pltpu.sync_copy(gb.at[pl.ds(0, 8)], acc.at[pl.ds(16, 8)])
    plsc.subcore_barrier()

    def _issue_inputs(sup, sidx, didx, nb, sem):
        r0 = base_rows + sup * SUBS
        pltpu.async_copy(src_h.at[pl.ds(r0, SUBS)], sidx, sem)
        pltpu.async_copy(dst_h.at[pl.ds(r0, SUBS)], didx, sem)
        pltpu.async_copy(norm_h.at[pl.ds(r0, SUBS)], nb, sem)

    def _wait_inputs(sidx, didx, nb, sem):
        pltpu.make_async_copy(src_h.at[pl.ds(0, SUBS)], sidx, sem).wait()
        pltpu.make_async_copy(dst_h.at[pl.ds(0, SUBS)], didx, sem).wait()
        pltpu.make_async_copy(norm_h.at[pl.ds(0, SUBS)], nb, sem).wait()

    def _super(sup, P, O, semO):
        sidxP, didxP, nbP = P
        sidxO, didxO, nbO = O
        @pl.when(sup < nsup - 1)
        def _():
            _issue_inputs(sup + 1, sidxO, didxO, nbO, semO)

        def _sub(r, _):
            rowv = jnp.zeros((16,), jnp.int32) + r
            m = didxP[r, pl.ds(0, 16)]
            for g in range(1, 8):
                m = jnp.minimum(m, didxP[r, pl.ds(g * 16, 16)])
            subhit = jnp.min(m) < NUM_GNBS
            @pl.when(subhit)
            def _():
                _sub_slow(r, rowv)
            return _

        def _sub_slow(r, rowv):
            for g in range(8):
                o = g * 16
                d16 = didxP[r, pl.ds(o, 16)]
                hit = jnp.min(d16) < NUM_GNBS
                @pl.when(hit)
                def _():
                    s16 = sidxP[r, pl.ds(o, 16)]
                    n16 = nbP[r, pl.ds(o, 16)]
                    si16[...] = s16
                    di16[...] = jnp.where(d16 < NUM_GNBS, d16, 20)
                    pltpu.sync_copy(h3_h.at[si16], gb)
                    for e in range(16):
                        colv = jnp.zeros((16,), jnp.int32) + (o + e)
                        spl = plsc.load_gather(nbP, [rowv, colv])
                        for q in range(4):
                            qo = q * 16
                            gb[e, pl.ds(qo, 16)] = gb[e, pl.ds(qo, 16)] * spl
                    pltpu.sync_copy(gb, acc.at[di16], add=True)
            return _
        lax.fori_loop(0, SUBS, _sub, None)

        @pl.when(sup < nsup - 1)
        def _():
            _wait_inputs(sidxO, didxO, nbO, semO)

    _issue_inputs(0, sidxA, didxA, nbA, sIA)
    _wait_inputs(sidxA, didxA, nbA, sIA)

    def _sloop(sup, _):
        @pl.when(sup % 2 == 0)
        def _():
            _super(sup, (sidxA, didxA, nbA), (sidxB, didxB, nbB), sIB)
        @pl.when(sup % 2 == 1)
        def _():
            _super(sup, (sidxB, didxB, nbB), (sidxA, didxA, nbA), sIA)
        return _
    lax.fori_loop(0, EPW // (SUBS * 128), _sloop, None)
    plsc.subcore_barrier()

    @pl.when(s == 0)
    def _():
        pltpu.sync_copy(acc.at[pl.ds(0, 16)], gb)
        pltpu.sync_copy(gb, agg_h.at[c, pl.ds(0, 16)])
        pltpu.sync_copy(acc.at[pl.ds(16, 8)], gb.at[pl.ds(0, 8)])
        pltpu.sync_copy(gb.at[pl.ds(0, 8)], agg_h.at[c, pl.ds(16, 8)])


def _mm1_body(agg_ref, w_ref, b_ref, o_ref):
    a = agg_ref[0] + agg_ref[1]
    z = jnp.dot(a, w_ref[...], preferred_element_type=jnp.float32) + b_ref[...]
    h = jnp.where(z >= 0, z, 0.01 * z)
    o_ref[0] = h[:, 0:32]
    o_ref[1] = h[:, 32:64]


def _mm1(aggp, w1p, b1):
    return pl.pallas_call(
        _mm1_body,
        grid=(NP // 512,),
        in_specs=[
            pl.BlockSpec((NC, 512, 16), lambda i: (0, i, 0)),
            pl.BlockSpec((16, 64), lambda i: (0, 0)),
            pl.BlockSpec((1, 64), lambda i: (0, 0)),
        ],
        out_specs=pl.BlockSpec((NC, 512, 32), lambda i: (0, i, 0)),
        out_shape=jax.ShapeDtypeStruct((NC, NP, 32), jnp.float32),
    )(aggp, w1p, b1.reshape(1, 64))


def _mm64_body(agg_ref, w_ref, b_ref, o_ref):
    w = w_ref[...]
    z = (jnp.dot(agg_ref[0], w[0:32, :], preferred_element_type=jnp.float32)
         + jnp.dot(agg_ref[1], w[32:64, :], preferred_element_type=jnp.float32)
         + b_ref[...])
    h = jnp.where(z >= 0, z, 0.01 * z)
    o_ref[0] = h[:, 0:32]
    o_ref[1] = h[:, 32:64]


def _mm64(agg, w, b):
    return pl.pallas_call(
        _mm64_body,
        grid=(NP // 512,),
        in_specs=[
            pl.BlockSpec((NC, 512, 32), lambda i: (0, i, 0)),
            pl.BlockSpec((64, 64), lambda i: (0, 0)),
            pl.BlockSpec((1, 64), lambda i: (0, 0)),
        ],
        out_specs=pl.BlockSpec((NC, 512, 32), lambda i: (0, i, 0)),
        out_shape=jax.ShapeDtypeStruct((NC, NP, 32), jnp.float32),
    )(agg, w, b.reshape(1, 64))


def _mm64f_body(agg_ref, w_ref, b_ref, of_ref):
    w = w_ref[...]
    z = (jnp.dot(agg_ref[0], w[0:32, :], preferred_element_type=jnp.float32)
         + jnp.dot(agg_ref[1], w[32:64, :], preferred_element_type=jnp.float32)
         + b_ref[...])
    of_ref[...] = jnp.where(z >= 0, z, 0.01 * z)


def _mm64f(agg, w, b):
    return pl.pallas_call(
        _mm64f_body,
        grid=(NP // 512,),
        in_specs=[
            pl.BlockSpec((NC, 512, 32), lambda i: (0, i, 0)),
            pl.BlockSpec((64, 64), lambda i: (0, 0)),
            pl.BlockSpec((1, 64), lambda i: (0, 0)),
        ],
        out_specs=pl.BlockSpec((512, 64), lambda i: (i, 0)),
        out_shape=jax.ShapeDtypeStruct((NP, 64), jnp.float32),
    )(agg, w, b.reshape(1, 64))


def _final_body(a_ref, w4_ref, b4_ref, wl_ref, bl_ref, o_ref):
    i = pl.program_id(0)
    o_ref[...] = jnp.full((512, 5), -jnp.inf, jnp.float32)
    @pl.when(i == 0)
    def _():
        a = a_ref[0] + a_ref[1]
        z4 = jnp.dot(a, w4_ref[...], preferred_element_type=jnp.float32) + b4_ref[...]
        h4 = jnp.where(z4 >= 0, z4, 0.01 * z4)
        z = jnp.dot(h4, wl_ref[...], preferred_element_type=jnp.float32) + bl_ref[...]
        rows = lax.broadcasted_iota(jnp.int32, (24, 5), 0)
        o_ref[0:24, :] = jnp.where(rows < NUM_GNBS, z, -jnp.inf)


def _final(agg4p, w4, b4, wl, bl):
    return pl.pallas_call(
        _final_body,
        grid=(98,),
        in_specs=[
            pl.BlockSpec((NC, 24, 64), lambda i: (0, 0, 0)),
            pl.BlockSpec((64, 64), lambda i: (0, 0)),
            pl.BlockSpec((1, 64), lambda i: (0, 0)),
            pl.BlockSpec((64, 5), lambda i: (0, 0)),
            pl.BlockSpec((1, 5), lambda i: (0, 0)),
        ],
        out_specs=pl.BlockSpec((512, 5), lambda i: (i, 0)),
        out_shape=jax.ShapeDtypeStruct((N, 5), jnp.float32),
    )(agg4p, w4, b4.reshape(1, 64), wl, bl.reshape(1, 5))


def kernel(x, edges, weights, W1, b1, W2, b2, W3, b3, W4, b4, Wl, bl):
    npad = EP - (E + N)
    loop = jnp.arange(N, dtype=jnp.int32)
    zpad = jnp.zeros((npad,), jnp.int32)
    src = jnp.concatenate([edges[0], loop, zpad])
    dst = jnp.concatenate([edges[1], loop, jnp.full((npad,), N - 1, jnp.int32)])
    ew = jnp.concatenate([weights, jnp.ones((N,), jnp.float32),
                          jnp.zeros((npad,), jnp.float32)])

    x_pad = jnp.zeros((NP, 16), jnp.float32).at[:N, :IN_F].set(x)
    w1p = jnp.zeros((16, H), jnp.float32).at[:IN_F].set(W1)

    src2 = src.reshape(EP // 128, 128)
    dst2 = dst.reshape(EP // 128, 128)
    ew2 = ew.reshape(EP // 128, 128)

    degp = _deg_kernel(dst2, ew2)
    dinv = _dinv(degp)
    norm2 = _norm_kernel(src2, dst2, ew2, dinv)
    aggp = _agg1_kernel(src2, dst2, norm2, x_pad)
    h1 = _mm1(aggp, w1p, b1)
    agg2 = _agg64_kernel(src2, dst2, norm2, h1.reshape(NC * NP, 32))
    h2 = _mm64(agg2, W2, b2)
    agg3 = _agg64_kernel(src2, dst2, norm2, h2.reshape(NC * NP, 32))
    h3full = _mm64f(agg3, W3, b3)
    agg4p = _agg19_kernel(src2, dst2, norm2, h3full)
    return _final(agg4p, W4, b4, Wl, bl)
